# baseline (speedup 1.0000x reference)
import functools

import jax
import jax.numpy as jnp
from jax import lax
from jax.experimental import pallas as pl
from jax.experimental.pallas import tpu as pltpu
from jax.experimental.pallas import tpu_sc as plsc

E = 160000
A = 10000
HP = 384
PW = 128
NP = 3
NT = 16
NW = 32
EPT = E // NT
EW = E // NW
L = 16

BS = 80
BC = 40
BCC = 80
NCC = -(-EW // BCC)

_mesh = plsc.VectorSubcoreMesh(core_axis_name="c", subcore_axis_name="s")
_f32 = jnp.float32


def _mm_pieces_out(a, w, bm):
    M, K = a.shape

    def body(a_ref, w_ref, o_ref):
        x = a_ref[...]
        for p in range(NP):
            o_ref[p] = jnp.dot(x, w_ref[:, p * PW:(p + 1) * PW],
                               preferred_element_type=_f32)

    return pl.pallas_call(
        body,
        grid=(M // bm,),
        in_specs=[
            pl.BlockSpec((bm, K), lambda i: (i, 0)),
            pl.BlockSpec((K, HP), lambda i: (0, 0)),
        ],
        out_specs=pl.BlockSpec((NP, bm, PW), lambda i: (0, i, 0)),
        out_shape=jax.ShapeDtypeStruct((NP, M, PW), _f32),
    )(a, w)


def _mm_pieces_both(h3, w, bm):
    _, M, _ = h3.shape

    def body(h_ref, w_ref, o_ref):
        res = jnp.dot(h_ref[0], w_ref[0:PW, :], preferred_element_type=_f32)
        for p in range(1, NP):
            res += jnp.dot(h_ref[p], w_ref[p * PW:(p + 1) * PW, :],
                           preferred_element_type=_f32)
        for p in range(NP):
            o_ref[p] = res[:, p * PW:(p + 1) * PW]

    return pl.pallas_call(
        body,
        grid=(M // bm,),
        in_specs=[
            pl.BlockSpec((NP, bm, PW), lambda i: (0, i, 0)),
            pl.BlockSpec((HP, HP), lambda i: (0, 0)),
        ],
        out_specs=pl.BlockSpec((NP, bm, PW), lambda i: (0, i, 0)),
        out_shape=jax.ShapeDtypeStruct((NP, M, PW), _f32),
    )(h3, w)


def _readout(af_pad, m3, wa, ba, wf1, bf1, wf2, bf2, bm):
    M = af_pad.shape[0]
    steps = M // bm

    def body(af_ref, m_ref, wa_ref, ba_ref, wf1_ref, bf1_ref, wf2_ref,
             bf2_ref, o_ref, acc_ref):
        i = pl.program_id(0)

        @pl.when(i == 0)
        def _():
            acc_ref[...] = jnp.zeros_like(acc_ref)

        ha = (jnp.dot(af_ref[...], wa_ref[0:80, :], preferred_element_type=_f32)
              + ba_ref[...])
        for p in range(NP):
            ha += jnp.dot(m_ref[p], wa_ref[80 + p * PW:80 + (p + 1) * PW, :],
                          preferred_element_type=_f32)
        ha = jnp.maximum(ha, 0.0)
        acc_ref[...] += jnp.sum(ha, axis=0, keepdims=True)

        o_ref[...] = jnp.zeros_like(o_ref)

        @pl.when(i == steps - 1)
        def _():
            mol = acc_ref[...] * (1.0 / M)
            hid = jnp.maximum(
                jnp.dot(mol, wf1_ref[...], preferred_element_type=_f32)
                + bf1_ref[...], 0.0)
            o_ref[...] = (jnp.dot(hid, wf2_ref[...], preferred_element_type=_f32)
                          + bf2_ref[...])

    return pl.pallas_call(
        body,
        grid=(steps,),
        in_specs=[
            pl.BlockSpec((bm, 80), lambda i: (i, 0)),
            pl.BlockSpec((NP, bm, PW), lambda i: (0, i, 0)),
            pl.BlockSpec((80 + HP, HP), lambda i: (0, 0)),
            pl.BlockSpec((1, HP), lambda i: (0, 0)),
            pl.BlockSpec((HP, HP), lambda i: (0, 0)),
            pl.BlockSpec((1, HP), lambda i: (0, 0)),
            pl.BlockSpec((HP, 128), lambda i: (0, 0)),
            pl.BlockSpec((1, 128), lambda i: (0, 0)),
        ],
        out_specs=pl.BlockSpec((1, 128), lambda i: (0, 0)),
        out_shape=jax.ShapeDtypeStruct((1, 128), _f32),
        scratch_shapes=[pltpu.VMEM((1, HP), _f32)],
    )(af_pad, m3, wa, ba, wf1, bf1, wf2, bf2)


def _relu_ab_minus_c(ab_ref, bb_ref, cb_ref, ob_ref, nrows):
    def row(i, _):
        for j in range(PW // L):
            s = pl.ds(j * L, L)
            x = ab_ref[i, s] + bb_ref[i, s] - cb_ref[i, s]
            ob_ref[i, s] = jnp.maximum(x, 0.0)
        return 0
    lax.fori_loop(0, nrows, row, 0)


def _zero_buf(z_ref, nrows):
    def row(i, _):
        for j in range(PW // L):
            z_ref[i, pl.ds(j * L, L)] = jnp.zeros((L,), _f32)
        return 0
    lax.fori_loop(0, nrows, row, 0)


def _zero_stripe(zb_ref, tbl_ref, t, rows):
    nch = A // rows
    for jj in range(-(-nch // NT)):
        cid = t + jj * NT

        @pl.when(cid < nch)
        def _():
            pltpu.sync_copy(zb_ref,
                            tbl_ref.at[pl.ds(pl.multiple_of(cid * rows, 8), rows)])


def _dump_stripe(tbl_ref, buf_ref, out_at, t, rows):
    nch = A // rows
    for jj in range(-(-nch // NT)):
        cid = t + jj * NT

        @pl.when(cid < nch)
        def _():
            off = pl.multiple_of(cid * rows, 8)
            pltpu.sync_copy(tbl_ref.at[pl.ds(off, rows)], buf_ref)
            pltpu.sync_copy(buf_ref, out_at.at[pl.ds(off, rows)])


BSS = 160
_NCH_ALL = E // BSS
_NCH_S = -(-_NCH_ALL // NT)


@functools.partial(
    pl.kernel,
    out_type=jax.ShapeDtypeStruct((NP, A, PW), _f32),
    mesh=_mesh,
    scratch_types=[
        pltpu.VMEM((BSS,), jnp.int32),
        pltpu.VMEM((BSS,), jnp.int32),
        pltpu.VMEM((BSS, PW), _f32),
        pltpu.VMEM((BSS, PW), _f32),
        pltpu.VMEM((BC, PW), _f32),
        pltpu.VMEM_SHARED((A, PW), _f32),
        pltpu.SemaphoreType.DMA,
        pltpu.SemaphoreType.DMA,
        pltpu.SemaphoreType.DMA,
        pltpu.SemaphoreType.DMA,
        pltpu.SemaphoreType.DMA,
        pltpu.SemaphoreType.DMA,
    ],
)
def _sc_scatter(hw3, src, agg3,
                si0, si1, rb0, rb1, zb, tbl,
                smi0, smi1, smr0, smr1, smw0, smw1):
    c = lax.axis_index("c")
    t = lax.axis_index("s")
    _zero_buf(zb, BC)
    sib = (si0, si1)
    rbb = (rb0, rb1)
    smi = (smi0, smi1)
    smr = (smr0, smr1)
    smw = (smw0, smw1)

    for rnd in range(2):
        piece = 2 * c + rnd
        active = piece < NP

        @pl.when(active)
        def _():
            _zero_stripe(zb, tbl, t, BC)
        plsc.subcore_barrier()

        @pl.when(active)
        def _():
            def cid(k):
                return t + k * NT

            def base(k):
                return pl.multiple_of(cid(k) * BSS, 8)

            def load(k, bank):
                pltpu.async_copy(hw3.at[piece, pl.ds(base(k), BSS)],
                                 rbb[bank], smr[bank])
                pltpu.async_copy(src.at[pl.ds(base(k), BSS)], sib[bank],
                                 smi[bank])

            def wait_load(k, bank):
                pltpu.make_async_copy(hw3.at[piece, pl.ds(base(k), BSS)],
                                      rbb[bank], smr[bank]).wait()
                pltpu.make_async_copy(src.at[pl.ds(base(k), BSS)], sib[bank],
                                      smi[bank]).wait()

            load(0, 0)
            load(1, 1)

            def step(k, bank):
                @pl.when(cid(k) < _NCH_ALL)
                def _():
                    wait_load(k, bank)
                    pltpu.async_copy(rbb[bank], tbl.at[sib[bank]], smw[bank],
                                     add=True)

            def drain_issue(k, bank):
                @pl.when(cid(k) < _NCH_ALL)
                def _():
                    pltpu.make_async_copy(rbb[bank], tbl.at[sib[bank]],
                                          smw[bank]).wait()

                    @pl.when(cid(k + 2) < _NCH_ALL)
                    def _():
                        load(k + 2, bank)

            def body(j, _):
                for bank in range(2):
                    step(2 * j + bank, bank)
                for bank in range(2):
                    drain_issue(2 * j + bank, bank)
                return 0

            lax.fori_loop(0, _NCH_S // 2, body, 0)
            if _NCH_S % 2 == 1:
                k = _NCH_S - 1

                @pl.when(cid(k) < _NCH_ALL)
                def _():
                    wait_load(k, 0)
                    pltpu.sync_copy(rbb[0], tbl.at[sib[0]], add=True)
        plsc.subcore_barrier()

        @pl.when(active)
        def _():
            _dump_stripe(tbl, zb, agg3.at[piece], t, BC)
            _zero_buf(zb, BC)
        plsc.subcore_barrier()


def _combine_scratch(n_idx, bc):
    return ([pltpu.VMEM((bc,), jnp.int32) for _ in range(2 * n_idx)] +
            [pltpu.VMEM((bc, PW), _f32) for _ in range(8)] +
            [pltpu.SemaphoreType.DMA for _ in range(6)])


def _combine_pipeline(nch, bc, bofs, piece, hw3, src, rev, hinit3, agg3,
                      sb, rb, hib, agb, rvb, ob, smi, smg, sms,
                      store_fn=None):
    agg_t = agg3.at[piece]
    hw_t = hw3.at[piece]

    def prefetch(k, bank):
        b = bofs(k)
        pltpu.async_copy(src.at[pl.ds(b, bc)], sb[bank], smi[bank])
        pltpu.async_copy(rev.at[pl.ds(b, bc)], rb[bank], smi[bank])

    def wait_prefetch(k, bank):
        b = bofs(k)
        pltpu.make_async_copy(src.at[pl.ds(b, bc)], sb[bank], smi[bank]).wait()
        pltpu.make_async_copy(rev.at[pl.ds(b, bc)], rb[bank], smi[bank]).wait()

    def issue(k, bank):
        wait_prefetch(k, bank)
        b = bofs(k)
        pltpu.async_copy(hinit3.at[piece, pl.ds(b, bc)], hib[bank], smg[bank])
        pltpu.async_copy(agg_t.at[sb[bank]], agb[bank], smg[bank])
        pltpu.async_copy(hw_t.at[rb[bank]], rvb[bank], smg[bank])

    def wait_gathers(k, bank):
        b = bofs(k)
        pltpu.make_async_copy(hinit3.at[piece, pl.ds(b, bc)], hib[bank],
                              smg[bank]).wait()
        pltpu.make_async_copy(agg_t.at[sb[bank]], agb[bank], smg[bank]).wait()
        pltpu.make_async_copy(hw_t.at[rb[bank]], rvb[bank], smg[bank]).wait()

    def finish(k, bank):
        wait_gathers(k, bank)

        @pl.when(k + 2 < nch)
        def _():
            prefetch(k + 2, bank)

        store_fn(k, bank)

        @pl.when(k + 2 < nch)
        def _():
            issue(k + 2, bank)

    prefetch(0, 0)
    prefetch(1, 1)
    issue(0, 0)
    issue(1, 1)

    def body(j, _):
        finish(2 * j, 0)
        finish(2 * j + 1, 1)
        return 0

    lax.fori_loop(0, nch // 2, body, 0)
    if nch % 2 == 1:
        finish(nch - 1, 0)


@functools.partial(
    pl.kernel,
    out_type=jax.ShapeDtypeStruct((NP, E, PW), _f32),
    mesh=_mesh,
    scratch_types=_combine_scratch(2, BCC),
)
def _sc_combine(hw3, src, rev, hinit3, agg3, h3,
                sb0, sb1, rb0, rb1,
                hib0, hib1, agb0, agb1, rvb0, rvb1, ob0, ob1,
                smi0, smi1, smg0, smg1, sms0, sms1):
    c = lax.axis_index("c")
    t = lax.axis_index("s")
    w = c * NT + t
    ebase = w * EW
    sb, rb = (sb0, sb1), (rb0, rb1)
    hib, agb, rvb, ob = (hib0, hib1), (agb0, agb1), (rvb0, rvb1), (ob0, ob1)
    smi, smg, sms = (smi0, smi1), (smg0, smg1), (sms0, sms1)

    def bofs(k):
        return ebase + pl.multiple_of(jnp.minimum(k * BCC, EW - BCC), 8)

    for piece in range(NP):
        out_t = h3.at[piece]

        def store(k, bank, out_t=out_t):
            b = bofs(k)

            @pl.when(k >= 2)
            def _():
                pltpu.make_async_copy(ob[bank], out_t.at[pl.ds(b, BCC)],
                                      sms[bank]).wait()

            _relu_ab_minus_c(hib[bank], agb[bank], rvb[bank], ob[bank], BCC)
            pltpu.async_copy(ob[bank], out_t.at[pl.ds(b, BCC)], sms[bank])

        _combine_pipeline(NCC, BCC, bofs, piece, hw3, src, rev, hinit3, agg3,
                          sb, rb, hib, agb, rvb, ob, smi, smg, sms,
                          store_fn=store)
        for bank in range(2):
            pltpu.make_async_copy(ob[bank], out_t.at[pl.ds(ebase, BCC)],
                                  sms[bank]).wait()


@functools.partial(
    pl.kernel,
    out_type=jax.ShapeDtypeStruct((NP, A, PW), _f32),
    mesh=_mesh,
    scratch_types=(
        [pltpu.VMEM((BC,), jnp.int32) for _ in range(6)] +
        [pltpu.VMEM((BC, PW), _f32) for _ in range(8)] +
        [pltpu.VMEM((BC, PW), _f32),
         pltpu.VMEM_SHARED((A, PW), _f32)] +
        [pltpu.SemaphoreType.DMA for _ in range(10)]
    ),
)
def _sc_combine_final(hw3, src, rev, dst, hinit3, agg3, m3,
                      sb0, sb1, rb0, rb1, db0, db1,
                      hib0, hib1, agb0, agb1, rvb0, rvb1, ob0, ob1,
                      zb, tbl,
                      smi0, smi1, smg0, smg1, sms0, sms1, smd0, smd1,
                      smm0, smm1):
    c = lax.axis_index("c")
    t = lax.axis_index("s")
    _zero_buf(zb, BC)
    sb, rb, db = (sb0, sb1), (rb0, rb1), (db0, db1)
    hib, agb, rvb, ob = (hib0, hib1), (agb0, agb1), (rvb0, rvb1), (ob0, ob1)
    smi, smg, sms = (smi0, smi1), (smg0, smg1), (sms0, sms1)
    smd, smm = (smd0, smd1), (smm0, smm1)
    nch = EPT // BC

    for rnd in range(2):
        piece = 2 * c + rnd
        active = piece < NP

        @pl.when(active)
        def _():
            _zero_stripe(zb, tbl, t, BC)
        plsc.subcore_barrier()

        @pl.when(active)
        def _():
            ebase = t * EPT

            def bofs(k):
                return ebase + k * BC

            def store(k, bank):
                b = bofs(k)

                @pl.when(k >= 2)
                def _():
                    pltpu.make_async_copy(ob[bank], tbl.at[db[bank]],
                                          smm[bank]).wait()

                pltpu.async_copy(dst.at[pl.ds(b, BC)], db[bank], smd[bank])
                _relu_ab_minus_c(hib[bank], agb[bank], rvb[bank], ob[bank], BC)
                pltpu.make_async_copy(dst.at[pl.ds(b, BC)], db[bank],
                                      smd[bank]).wait()
                pltpu.async_copy(ob[bank], tbl.at[db[bank]], smm[bank],
                                 add=True)

            _combine_pipeline(nch, BC, bofs, piece, hw3, src, rev, hinit3,
                              agg3, sb, rb, hib, agb, rvb, ob, smi, smg, sms,
                              store_fn=store)
            for bank in range(2):
                pltpu.make_async_copy(ob[bank], tbl.at[db[bank]],
                                      smm[bank]).wait()
        plsc.subcore_barrier()

        @pl.when(active)
        def _():
            _dump_stripe(tbl, zb, m3.at[piece], t, BC)
            _zero_buf(zb, BC)
        plsc.subcore_barrier()


@functools.partial(
    pl.kernel,
    out_type=jax.ShapeDtypeStruct((NP, E, PW), _f32),
    mesh=_mesh,
    scratch_types=(
        [pltpu.VMEM((BCC,), jnp.int32) for _ in range(2)] +
        [pltpu.VMEM((BCC, PW), _f32) for _ in range(6)] +
        [pltpu.SemaphoreType.DMA for _ in range(6)]
    ),
)
def _sc_hinit(p3, q3, src, out,
              sb0, sb1, pb0, pb1, qb0, qb1, ob0, ob1,
              smi0, smi1, smg0, smg1, sms0, sms1):
    c = lax.axis_index("c")
    t = lax.axis_index("s")
    w = c * NT + t
    ebase = w * EW
    nch = NCC
    sb = (sb0, sb1)
    pb, qb, ob = (pb0, pb1), (qb0, qb1), (ob0, ob1)
    smi, smg, sms = (smi0, smi1), (smg0, smg1), (sms0, sms1)

    def bofs(k):
        return ebase + pl.multiple_of(jnp.minimum(k * BCC, EW - BCC), 8)

    for piece in range(NP):
        p_t = p3.at[piece]
        out_t = out.at[piece]

        def prefetch(k, bank):
            pltpu.async_copy(src.at[pl.ds(bofs(k), BCC)], sb[bank],
                             smi[bank])

        def issue(k, bank):
            b = bofs(k)
            pltpu.make_async_copy(src.at[pl.ds(b, BCC)], sb[bank],
                                  smi[bank]).wait()
            pltpu.async_copy(p_t.at[sb[bank]], pb[bank], smg[bank])
            pltpu.async_copy(q3.at[piece, pl.ds(b, BCC)], qb[bank], smg[bank])

        def finish(k, bank):
            b = bofs(k)
            pltpu.make_async_copy(p_t.at[sb[bank]], pb[bank], smg[bank]).wait()
            pltpu.make_async_copy(q3.at[piece, pl.ds(b, BCC)], qb[bank],
                                  smg[bank]).wait()

            @pl.when(k + 2 < nch)
            def _():
                prefetch(k + 2, bank)

            @pl.when(k >= 2)
            def _():
                pltpu.make_async_copy(ob[bank], out_t.at[pl.ds(b, BCC)],
                                      sms[bank]).wait()

            def row(i, _):
                for j in range(PW // L):
                    s = pl.ds(j * L, L)
                    ob[bank][i, s] = jnp.maximum(pb[bank][i, s]
                                                 + qb[bank][i, s], 0.0)
                return 0
            lax.fori_loop(0, BCC, row, 0)
            pltpu.async_copy(ob[bank], out_t.at[pl.ds(b, BCC)], sms[bank])

            @pl.when(k + 2 < nch)
            def _():
                issue(k + 2, bank)

        prefetch(0, 0)
        prefetch(1, 1)
        issue(0, 0)
        issue(1, 1)

        def body(j, _):
            finish(2 * j, 0)
            finish(2 * j + 1, 1)
            return 0

        lax.fori_loop(0, nch // 2, body, 0)
        if nch % 2 == 1:
            finish(nch - 1, 0)
        for bank in range(2):
            pltpu.make_async_copy(ob[bank], out_t.at[pl.ds(ebase, BCC)],
                                  sms[bank]).wait()


def _pad2(w, rows, cols):
    return jnp.pad(w, ((0, rows - w.shape[0]), (0, cols - w.shape[1])))


def kernel(atom_feats, bond_feats, src, dst, rev_idx, n_atoms,
           W_i, W_m0, W_m1, W_m2, W_a, b_a, W_f1, b_f1, W_f2, b_f2):
    del n_atoms
    src = src.astype(jnp.int32)
    dst = dst.astype(jnp.int32)
    rev_idx = rev_idx.astype(jnp.int32)

    af_pad = jnp.pad(atom_feats, ((0, 0), (0, 80 - 73)))
    bf_pad = jnp.pad(bond_feats, ((0, 0), (0, 16 - 13)))
    wi_top = _pad2(W_i[:73], 80, HP)
    wi_bot = _pad2(W_i[73:], 16, HP)
    wm = [_pad2(w, HP, HP) for w in (W_m0, W_m1, W_m2)]
    wa = jnp.concatenate([_pad2(W_a[:73], 80, HP), _pad2(W_a[73:], HP, HP)],
                         axis=0)
    ba = _pad2(b_a[None, :], 1, HP)
    wf1 = _pad2(W_f1, HP, HP)
    bf1 = _pad2(b_f1[None, :], 1, HP)
    wf2 = _pad2(W_f2, HP, 128)
    bf2 = _pad2(b_f2[None, :], 1, 128)

    p3 = _mm_pieces_out(af_pad, wi_top, 2000)
    q3 = _mm_pieces_out(bf_pad, wi_bot, 2000)
    hinit3 = _sc_hinit(p3, q3, src)

    h3 = hinit3
    for li in range(3):
        hw3 = _mm_pieces_both(h3, wm[li], 2000)
        agg3 = _sc_scatter(hw3, src)
        if li < 2:
            h3 = _sc_combine(hw3, src, rev_idx, hinit3, agg3)
        else:
            m3 = _sc_combine_final(hw3, src, rev_idx, dst, hinit3, agg3)

    out_pad = _readout(af_pad, m3, wa, ba, wf1, bf1, wf2, bf2, 2000)
    return out_pad[:, :3]


# speedup vs baseline: 1.0264x; 1.0264x over previous
import functools

import jax
import jax.numpy as jnp
from jax import lax
from jax.experimental import pallas as pl
from jax.experimental.pallas import tpu as pltpu
from jax.experimental.pallas import tpu_sc as plsc

E = 160000
A = 10000
HP = 384
PW = 128
NP = 3
NT = 16
NW = 32
EPT = E // NT
EW = E // NW
L = 16

BS = 80
BC = 40
BCC = 80
NCC = -(-EW // BCC)

_mesh = plsc.VectorSubcoreMesh(core_axis_name="c", subcore_axis_name="s")
_f32 = jnp.float32


def _mm_pieces_out(a, w, bm):
    M, K = a.shape

    def body(a_ref, w_ref, o_ref):
        x = a_ref[...]
        for p in range(NP):
            o_ref[p] = jnp.dot(x, w_ref[:, p * PW:(p + 1) * PW],
                               preferred_element_type=_f32)

    return pl.pallas_call(
        body,
        grid=(M // bm,),
        in_specs=[
            pl.BlockSpec((bm, K), lambda i: (i, 0)),
            pl.BlockSpec((K, HP), lambda i: (0, 0)),
        ],
        out_specs=pl.BlockSpec((NP, bm, PW), lambda i: (0, i, 0)),
        out_shape=jax.ShapeDtypeStruct((NP, M, PW), _f32),
    )(a, w)


def _mm_pieces_both(h3, w, bm):
    _, M, _ = h3.shape

    def body(h_ref, w_ref, o_ref):
        hcat = jnp.concatenate([h_ref[0], h_ref[1], h_ref[2]], axis=1)
        res = jnp.dot(hcat, w_ref[...], preferred_element_type=_f32)
        for p in range(NP):
            o_ref[p] = res[:, p * PW:(p + 1) * PW]

    return pl.pallas_call(
        body,
        grid=(M // bm,),
        in_specs=[
            pl.BlockSpec((NP, bm, PW), lambda i: (0, i, 0)),
            pl.BlockSpec((HP, HP), lambda i: (0, 0)),
        ],
        out_specs=pl.BlockSpec((NP, bm, PW), lambda i: (0, i, 0)),
        out_shape=jax.ShapeDtypeStruct((NP, M, PW), _f32),
    )(h3, w)


def _readout(af_pad, m3, wa, ba, wf1, bf1, wf2, bf2, bm):
    M = af_pad.shape[0]
    steps = M // bm

    def body(af_ref, m_ref, wa_ref, ba_ref, wf1_ref, bf1_ref, wf2_ref,
             bf2_ref, o_ref, acc_ref):
        i = pl.program_id(0)

        @pl.when(i == 0)
        def _():
            acc_ref[...] = jnp.zeros_like(acc_ref)

        ha = (jnp.dot(af_ref[...], wa_ref[0:80, :], preferred_element_type=_f32)
              + ba_ref[...])
        for p in range(NP):
            ha += jnp.dot(m_ref[p], wa_ref[80 + p * PW:80 + (p + 1) * PW, :],
                          preferred_element_type=_f32)
        ha = jnp.maximum(ha, 0.0)
        acc_ref[...] += jnp.sum(ha, axis=0, keepdims=True)

        o_ref[...] = jnp.zeros_like(o_ref)

        @pl.when(i == steps - 1)
        def _():
            mol = acc_ref[...] * (1.0 / M)
            hid = jnp.maximum(
                jnp.dot(mol, wf1_ref[...], preferred_element_type=_f32)
                + bf1_ref[...], 0.0)
            o_ref[...] = (jnp.dot(hid, wf2_ref[...], preferred_element_type=_f32)
                          + bf2_ref[...])

    return pl.pallas_call(
        body,
        grid=(steps,),
        in_specs=[
            pl.BlockSpec((bm, 80), lambda i: (i, 0)),
            pl.BlockSpec((NP, bm, PW), lambda i: (0, i, 0)),
            pl.BlockSpec((80 + HP, HP), lambda i: (0, 0)),
            pl.BlockSpec((1, HP), lambda i: (0, 0)),
            pl.BlockSpec((HP, HP), lambda i: (0, 0)),
            pl.BlockSpec((1, HP), lambda i: (0, 0)),
            pl.BlockSpec((HP, 128), lambda i: (0, 0)),
            pl.BlockSpec((1, 128), lambda i: (0, 0)),
        ],
        out_specs=pl.BlockSpec((1, 128), lambda i: (0, 0)),
        out_shape=jax.ShapeDtypeStruct((1, 128), _f32),
        scratch_shapes=[pltpu.VMEM((1, HP), _f32)],
    )(af_pad, m3, wa, ba, wf1, bf1, wf2, bf2)


def _relu_ab_minus_c(ab_ref, bb_ref, cb_ref, ob_ref, nrows):
    def row(i, _):
        for j in range(PW // L):
            s = pl.ds(j * L, L)
            x = ab_ref[i, s] + bb_ref[i, s] - cb_ref[i, s]
            ob_ref[i, s] = jnp.maximum(x, 0.0)
        return 0
    lax.fori_loop(0, nrows, row, 0)


def _zero_buf(z_ref, nrows):
    def row(i, _):
        for j in range(PW // L):
            z_ref[i, pl.ds(j * L, L)] = jnp.zeros((L,), _f32)
        return 0
    lax.fori_loop(0, nrows, row, 0)


def _zero_stripe(zb_ref, tbl_ref, t, rows):
    nch = A // rows
    for jj in range(-(-nch // NT)):
        cid = t + jj * NT

        @pl.when(cid < nch)
        def _():
            pltpu.sync_copy(zb_ref,
                            tbl_ref.at[pl.ds(pl.multiple_of(cid * rows, 8), rows)])


def _dump_stripe(tbl_ref, buf_ref, out_at, t, rows):
    nch = A // rows
    for jj in range(-(-nch // NT)):
        cid = t + jj * NT

        @pl.when(cid < nch)
        def _():
            off = pl.multiple_of(cid * rows, 8)
            pltpu.sync_copy(tbl_ref.at[pl.ds(off, rows)], buf_ref)
            pltpu.sync_copy(buf_ref, out_at.at[pl.ds(off, rows)])


BSS = 160
_NCH_ALL = E // BSS
_NCH_S = -(-_NCH_ALL // NT)


@functools.partial(
    pl.kernel,
    out_type=jax.ShapeDtypeStruct((NP, A, PW), _f32),
    mesh=_mesh,
    scratch_types=[
        pltpu.VMEM((BSS,), jnp.int32),
        pltpu.VMEM((BSS,), jnp.int32),
        pltpu.VMEM((BSS, PW), _f32),
        pltpu.VMEM((BSS, PW), _f32),
        pltpu.VMEM((BC, PW), _f32),
        pltpu.VMEM_SHARED((A, PW), _f32),
        pltpu.SemaphoreType.DMA,
        pltpu.SemaphoreType.DMA,
        pltpu.SemaphoreType.DMA,
        pltpu.SemaphoreType.DMA,
        pltpu.SemaphoreType.DMA,
        pltpu.SemaphoreType.DMA,
    ],
)
def _sc_scatter(hw3, src, agg3,
                si0, si1, rb0, rb1, zb, tbl,
                smi0, smi1, smr0, smr1, smw0, smw1):
    c = lax.axis_index("c")
    t = lax.axis_index("s")
    _zero_buf(zb, BC)
    sib = (si0, si1)
    rbb = (rb0, rb1)
    smi = (smi0, smi1)
    smr = (smr0, smr1)
    smw = (smw0, smw1)

    for rnd in range(2):
        piece = 2 * c + rnd
        active = piece < NP

        @pl.when(active)
        def _():
            _zero_stripe(zb, tbl, t, BC)
        plsc.subcore_barrier()

        @pl.when(active)
        def _():
            def cid(k):
                return t + k * NT

            def base(k):
                return pl.multiple_of(cid(k) * BSS, 8)

            def load(k, bank):
                pltpu.async_copy(hw3.at[piece, pl.ds(base(k), BSS)],
                                 rbb[bank], smr[bank])
                pltpu.async_copy(src.at[pl.ds(base(k), BSS)], sib[bank],
                                 smi[bank])

            def wait_load(k, bank):
                pltpu.make_async_copy(hw3.at[piece, pl.ds(base(k), BSS)],
                                      rbb[bank], smr[bank]).wait()
                pltpu.make_async_copy(src.at[pl.ds(base(k), BSS)], sib[bank],
                                      smi[bank]).wait()

            load(0, 0)
            load(1, 1)

            def step(k, bank):
                @pl.when(cid(k) < _NCH_ALL)
                def _():
                    wait_load(k, bank)
                    pltpu.async_copy(rbb[bank], tbl.at[sib[bank]], smw[bank],
                                     add=True)

            def drain_issue(k, bank):
                @pl.when(cid(k) < _NCH_ALL)
                def _():
                    pltpu.make_async_copy(rbb[bank], tbl.at[sib[bank]],
                                          smw[bank]).wait()

                    @pl.when(cid(k + 2) < _NCH_ALL)
                    def _():
                        load(k + 2, bank)

            def body(j, _):
                for bank in range(2):
                    step(2 * j + bank, bank)
                for bank in range(2):
                    drain_issue(2 * j + bank, bank)
                return 0

            lax.fori_loop(0, _NCH_S // 2, body, 0)
            if _NCH_S % 2 == 1:
                k = _NCH_S - 1

                @pl.when(cid(k) < _NCH_ALL)
                def _():
                    wait_load(k, 0)
                    pltpu.sync_copy(rbb[0], tbl.at[sib[0]], add=True)
        plsc.subcore_barrier()

        @pl.when(active)
        def _():
            _dump_stripe(tbl, zb, agg3.at[piece], t, BC)
            _zero_buf(zb, BC)
        plsc.subcore_barrier()


def _combine_scratch(n_idx, bc):
    return ([pltpu.VMEM((bc,), jnp.int32) for _ in range(2 * n_idx)] +
            [pltpu.VMEM((bc, PW), _f32) for _ in range(8)] +
            [pltpu.SemaphoreType.DMA for _ in range(6)])


def _combine_pipeline(nch, bc, bofs, piece, hw3, src, rev, hinit3, agg3,
                      sb, rb, hib, agb, rvb, ob, smi, smg, sms,
                      store_fn=None):
    agg_t = agg3.at[piece]
    hw_t = hw3.at[piece]

    def prefetch(k, bank):
        b = bofs(k)
        pltpu.async_copy(src.at[pl.ds(b, bc)], sb[bank], smi[bank])
        pltpu.async_copy(rev.at[pl.ds(b, bc)], rb[bank], smi[bank])

    def wait_prefetch(k, bank):
        b = bofs(k)
        pltpu.make_async_copy(src.at[pl.ds(b, bc)], sb[bank], smi[bank]).wait()
        pltpu.make_async_copy(rev.at[pl.ds(b, bc)], rb[bank], smi[bank]).wait()

    def issue(k, bank):
        wait_prefetch(k, bank)
        b = bofs(k)
        pltpu.async_copy(hinit3.at[piece, pl.ds(b, bc)], hib[bank], smg[bank])
        pltpu.async_copy(agg_t.at[sb[bank]], agb[bank], smg[bank])
        pltpu.async_copy(hw_t.at[rb[bank]], rvb[bank], smg[bank])

    def wait_gathers(k, bank):
        b = bofs(k)
        pltpu.make_async_copy(hinit3.at[piece, pl.ds(b, bc)], hib[bank],
                              smg[bank]).wait()
        pltpu.make_async_copy(agg_t.at[sb[bank]], agb[bank], smg[bank]).wait()
        pltpu.make_async_copy(hw_t.at[rb[bank]], rvb[bank], smg[bank]).wait()

    def finish(k, bank):
        wait_gathers(k, bank)

        @pl.when(k + 2 < nch)
        def _():
            prefetch(k + 2, bank)

        store_fn(k, bank)

        @pl.when(k + 2 < nch)
        def _():
            issue(k + 2, bank)

    prefetch(0, 0)
    prefetch(1, 1)
    issue(0, 0)
    issue(1, 1)

    def body(j, _):
        finish(2 * j, 0)
        finish(2 * j + 1, 1)
        return 0

    lax.fori_loop(0, nch // 2, body, 0)
    if nch % 2 == 1:
        finish(nch - 1, 0)


@functools.partial(
    pl.kernel,
    out_type=jax.ShapeDtypeStruct((NP, E, PW), _f32),
    mesh=_mesh,
    scratch_types=_combine_scratch(2, BCC),
)
def _sc_combine(hw3, src, rev, hinit3, agg3, h3,
                sb0, sb1, rb0, rb1,
                hib0, hib1, agb0, agb1, rvb0, rvb1, ob0, ob1,
                smi0, smi1, smg0, smg1, sms0, sms1):
    c = lax.axis_index("c")
    t = lax.axis_index("s")
    w = c * NT + t
    ebase = w * EW
    sb, rb = (sb0, sb1), (rb0, rb1)
    hib, agb, rvb, ob = (hib0, hib1), (agb0, agb1), (rvb0, rvb1), (ob0, ob1)
    smi, smg, sms = (smi0, smi1), (smg0, smg1), (sms0, sms1)

    def bofs(k):
        return ebase + pl.multiple_of(jnp.minimum(k * BCC, EW - BCC), 8)

    for piece in range(NP):
        out_t = h3.at[piece]

        def store(k, bank, out_t=out_t):
            b = bofs(k)

            @pl.when(k >= 2)
            def _():
                pltpu.make_async_copy(ob[bank], out_t.at[pl.ds(b, BCC)],
                                      sms[bank]).wait()

            _relu_ab_minus_c(hib[bank], agb[bank], rvb[bank], ob[bank], BCC)
            pltpu.async_copy(ob[bank], out_t.at[pl.ds(b, BCC)], sms[bank])

        _combine_pipeline(NCC, BCC, bofs, piece, hw3, src, rev, hinit3, agg3,
                          sb, rb, hib, agb, rvb, ob, smi, smg, sms,
                          store_fn=store)
        for bank in range(2):
            pltpu.make_async_copy(ob[bank], out_t.at[pl.ds(ebase, BCC)],
                                  sms[bank]).wait()


@functools.partial(
    pl.kernel,
    out_type=jax.ShapeDtypeStruct((NP, A, PW), _f32),
    mesh=_mesh,
    scratch_types=(
        [pltpu.VMEM((BC,), jnp.int32) for _ in range(6)] +
        [pltpu.VMEM((BC, PW), _f32) for _ in range(8)] +
        [pltpu.VMEM((BC, PW), _f32),
         pltpu.VMEM_SHARED((A, PW), _f32)] +
        [pltpu.SemaphoreType.DMA for _ in range(10)]
    ),
)
def _sc_combine_final(hw3, src, rev, dst, hinit3, agg3, m3,
                      sb0, sb1, rb0, rb1, db0, db1,
                      hib0, hib1, agb0, agb1, rvb0, rvb1, ob0, ob1,
                      zb, tbl,
                      smi0, smi1, smg0, smg1, sms0, sms1, smd0, smd1,
                      smm0, smm1):
    c = lax.axis_index("c")
    t = lax.axis_index("s")
    _zero_buf(zb, BC)
    sb, rb, db = (sb0, sb1), (rb0, rb1), (db0, db1)
    hib, agb, rvb, ob = (hib0, hib1), (agb0, agb1), (rvb0, rvb1), (ob0, ob1)
    smi, smg, sms = (smi0, smi1), (smg0, smg1), (sms0, sms1)
    smd, smm = (smd0, smd1), (smm0, smm1)
    nch = EPT // BC

    for rnd in range(2):
        piece = 2 * c + rnd
        active = piece < NP

        @pl.when(active)
        def _():
            _zero_stripe(zb, tbl, t, BC)
        plsc.subcore_barrier()

        @pl.when(active)
        def _():
            ebase = t * EPT

            def bofs(k):
                return ebase + k * BC

            def store(k, bank):
                b = bofs(k)

                @pl.when(k >= 2)
                def _():
                    pltpu.make_async_copy(ob[bank], tbl.at[db[bank]],
                                          smm[bank]).wait()

                pltpu.async_copy(dst.at[pl.ds(b, BC)], db[bank], smd[bank])
                _relu_ab_minus_c(hib[bank], agb[bank], rvb[bank], ob[bank], BC)
                pltpu.make_async_copy(dst.at[pl.ds(b, BC)], db[bank],
                                      smd[bank]).wait()
                pltpu.async_copy(ob[bank], tbl.at[db[bank]], smm[bank],
                                 add=True)

            _combine_pipeline(nch, BC, bofs, piece, hw3, src, rev, hinit3,
                              agg3, sb, rb, hib, agb, rvb, ob, smi, smg, sms,
                              store_fn=store)
            for bank in range(2):
                pltpu.make_async_copy(ob[bank], tbl.at[db[bank]],
                                      smm[bank]).wait()
        plsc.subcore_barrier()

        @pl.when(active)
        def _():
            _dump_stripe(tbl, zb, m3.at[piece], t, BC)
            _zero_buf(zb, BC)
        plsc.subcore_barrier()


@functools.partial(
    pl.kernel,
    out_type=jax.ShapeDtypeStruct((NP, E, PW), _f32),
    mesh=_mesh,
    scratch_types=(
        [pltpu.VMEM((BCC,), jnp.int32) for _ in range(2)] +
        [pltpu.VMEM((BCC, PW), _f32) for _ in range(6)] +
        [pltpu.SemaphoreType.DMA for _ in range(6)]
    ),
)
def _sc_hinit(p3, q3, src, out,
              sb0, sb1, pb0, pb1, qb0, qb1, ob0, ob1,
              smi0, smi1, smg0, smg1, sms0, sms1):
    c = lax.axis_index("c")
    t = lax.axis_index("s")
    w = c * NT + t
    ebase = w * EW
    nch = NCC
    sb = (sb0, sb1)
    pb, qb, ob = (pb0, pb1), (qb0, qb1), (ob0, ob1)
    smi, smg, sms = (smi0, smi1), (smg0, smg1), (sms0, sms1)

    def bofs(k):
        return ebase + pl.multiple_of(jnp.minimum(k * BCC, EW - BCC), 8)

    for piece in range(NP):
        p_t = p3.at[piece]
        out_t = out.at[piece]

        def prefetch(k, bank):
            pltpu.async_copy(src.at[pl.ds(bofs(k), BCC)], sb[bank],
                             smi[bank])

        def issue(k, bank):
            b = bofs(k)
            pltpu.make_async_copy(src.at[pl.ds(b, BCC)], sb[bank],
                                  smi[bank]).wait()
            pltpu.async_copy(p_t.at[sb[bank]], pb[bank], smg[bank])
            pltpu.async_copy(q3.at[piece, pl.ds(b, BCC)], qb[bank], smg[bank])

        def finish(k, bank):
            b = bofs(k)
            pltpu.make_async_copy(p_t.at[sb[bank]], pb[bank], smg[bank]).wait()
            pltpu.make_async_copy(q3.at[piece, pl.ds(b, BCC)], qb[bank],
                                  smg[bank]).wait()

            @pl.when(k + 2 < nch)
            def _():
                prefetch(k + 2, bank)

            @pl.when(k >= 2)
            def _():
                pltpu.make_async_copy(ob[bank], out_t.at[pl.ds(b, BCC)],
                                      sms[bank]).wait()

            def row(i, _):
                for j in range(PW // L):
                    s = pl.ds(j * L, L)
                    ob[bank][i, s] = jnp.maximum(pb[bank][i, s]
                                                 + qb[bank][i, s], 0.0)
                return 0
            lax.fori_loop(0, BCC, row, 0)
            pltpu.async_copy(ob[bank], out_t.at[pl.ds(b, BCC)], sms[bank])

            @pl.when(k + 2 < nch)
            def _():
                issue(k + 2, bank)

        prefetch(0, 0)
        prefetch(1, 1)
        issue(0, 0)
        issue(1, 1)

        def body(j, _):
            finish(2 * j, 0)
            finish(2 * j + 1, 1)
            return 0

        lax.fori_loop(0, nch // 2, body, 0)
        if nch % 2 == 1:
            finish(nch - 1, 0)
        for bank in range(2):
            pltpu.make_async_copy(ob[bank], out_t.at[pl.ds(ebase, BCC)],
                                  sms[bank]).wait()


def _pad2(w, rows, cols):
    return jnp.pad(w, ((0, rows - w.shape[0]), (0, cols - w.shape[1])))


def kernel(atom_feats, bond_feats, src, dst, rev_idx, n_atoms,
           W_i, W_m0, W_m1, W_m2, W_a, b_a, W_f1, b_f1, W_f2, b_f2):
    del n_atoms
    src = src.astype(jnp.int32)
    dst = dst.astype(jnp.int32)
    rev_idx = rev_idx.astype(jnp.int32)

    af_pad = jnp.pad(atom_feats, ((0, 0), (0, 80 - 73)))
    bf_pad = jnp.pad(bond_feats, ((0, 0), (0, 16 - 13)))
    wi_top = _pad2(W_i[:73], 80, HP)
    wi_bot = _pad2(W_i[73:], 16, HP)
    wm = [_pad2(w, HP, HP) for w in (W_m0, W_m1, W_m2)]
    wa = jnp.concatenate([_pad2(W_a[:73], 80, HP), _pad2(W_a[73:], HP, HP)],
                         axis=0)
    ba = _pad2(b_a[None, :], 1, HP)
    wf1 = _pad2(W_f1, HP, HP)
    bf1 = _pad2(b_f1[None, :], 1, HP)
    wf2 = _pad2(W_f2, HP, 128)
    bf2 = _pad2(b_f2[None, :], 1, 128)

    p3 = _mm_pieces_out(af_pad, wi_top, 2000)
    q3 = _mm_pieces_out(bf_pad, wi_bot, 2000)
    hinit3 = _sc_hinit(p3, q3, src)

    h3 = hinit3
    for li in range(3):
        hw3 = _mm_pieces_both(h3, wm[li], 4000)
        agg3 = _sc_scatter(hw3, src)
        if li < 2:
            h3 = _sc_combine(hw3, src, rev_idx, hinit3, agg3)
        else:
            m3 = _sc_combine_final(hw3, src, rev_idx, dst, hinit3, agg3)

    out_pad = _readout(af_pad, m3, wa, ba, wf1, bf1, wf2, bf2, 2000)
    return out_pad[:, :3]


# speedup vs baseline: 1.0317x; 1.0051x over previous
import functools

import jax
import jax.numpy as jnp
from jax import lax
from jax.experimental import pallas as pl
from jax.experimental.pallas import tpu as pltpu
from jax.experimental.pallas import tpu_sc as plsc

E = 160000
A = 10000
HP = 384
PW = 128
NP = 3
NT = 16
NW = 32
EPT = E // NT
EW = E // NW
L = 16

BS = 80
BC = 40
BCC = 80
NCC = -(-EW // BCC)

_mesh = plsc.VectorSubcoreMesh(core_axis_name="c", subcore_axis_name="s")
_f32 = jnp.float32


def _mm_pieces_out(a, w, bm):
    M, K = a.shape

    def body(a_ref, w_ref, o_ref):
        x = a_ref[...]
        for p in range(NP):
            o_ref[p] = jnp.dot(x, w_ref[:, p * PW:(p + 1) * PW],
                               preferred_element_type=_f32)

    return pl.pallas_call(
        body,
        grid=(M // bm,),
        in_specs=[
            pl.BlockSpec((bm, K), lambda i: (i, 0)),
            pl.BlockSpec((K, HP), lambda i: (0, 0)),
        ],
        out_specs=pl.BlockSpec((NP, bm, PW), lambda i: (0, i, 0)),
        out_shape=jax.ShapeDtypeStruct((NP, M, PW), _f32),
    )(a, w)


def _mm_pieces_both(h3, w, bm):
    _, M, _ = h3.shape

    def body(h_ref, w_ref, o_ref):
        hcat = jnp.concatenate([h_ref[0], h_ref[1], h_ref[2]], axis=1)
        res = jnp.dot(hcat, w_ref[...], preferred_element_type=_f32)
        for p in range(NP):
            o_ref[p] = res[:, p * PW:(p + 1) * PW]

    return pl.pallas_call(
        body,
        grid=(M // bm,),
        in_specs=[
            pl.BlockSpec((NP, bm, PW), lambda i: (0, i, 0)),
            pl.BlockSpec((HP, HP), lambda i: (0, 0)),
        ],
        out_specs=pl.BlockSpec((NP, bm, PW), lambda i: (0, i, 0)),
        out_shape=jax.ShapeDtypeStruct((NP, M, PW), _f32),
    )(h3, w)


def _readout(af_pad, m3, wa, ba, wf1, bf1, wf2, bf2, bm):
    M = af_pad.shape[0]
    steps = M // bm

    def body(af_ref, m_ref, wa_ref, ba_ref, wf1_ref, bf1_ref, wf2_ref,
             bf2_ref, o_ref, acc_ref):
        i = pl.program_id(0)

        @pl.when(i == 0)
        def _():
            acc_ref[...] = jnp.zeros_like(acc_ref)

        ha = (jnp.dot(af_ref[...], wa_ref[0:80, :], preferred_element_type=_f32)
              + ba_ref[...])
        for p in range(NP):
            ha += jnp.dot(m_ref[p], wa_ref[80 + p * PW:80 + (p + 1) * PW, :],
                          preferred_element_type=_f32)
        ha = jnp.maximum(ha, 0.0)
        acc_ref[...] += jnp.sum(ha, axis=0, keepdims=True)

        o_ref[...] = jnp.zeros_like(o_ref)

        @pl.when(i == steps - 1)
        def _():
            mol = acc_ref[...] * (1.0 / M)
            hid = jnp.maximum(
                jnp.dot(mol, wf1_ref[...], preferred_element_type=_f32)
                + bf1_ref[...], 0.0)
            o_ref[...] = (jnp.dot(hid, wf2_ref[...], preferred_element_type=_f32)
                          + bf2_ref[...])

    return pl.pallas_call(
        body,
        grid=(steps,),
        in_specs=[
            pl.BlockSpec((bm, 80), lambda i: (i, 0)),
            pl.BlockSpec((NP, bm, PW), lambda i: (0, i, 0)),
            pl.BlockSpec((80 + HP, HP), lambda i: (0, 0)),
            pl.BlockSpec((1, HP), lambda i: (0, 0)),
            pl.BlockSpec((HP, HP), lambda i: (0, 0)),
            pl.BlockSpec((1, HP), lambda i: (0, 0)),
            pl.BlockSpec((HP, 128), lambda i: (0, 0)),
            pl.BlockSpec((1, 128), lambda i: (0, 0)),
        ],
        out_specs=pl.BlockSpec((1, 128), lambda i: (0, 0)),
        out_shape=jax.ShapeDtypeStruct((1, 128), _f32),
        scratch_shapes=[pltpu.VMEM((1, HP), _f32)],
    )(af_pad, m3, wa, ba, wf1, bf1, wf2, bf2)


def _relu_ab_minus_c(ab_ref, bb_ref, cb_ref, ob_ref, nrows):
    def row(i, _):
        for j in range(PW // L):
            s = pl.ds(j * L, L)
            x = ab_ref[i, s] + bb_ref[i, s] - cb_ref[i, s]
            ob_ref[i, s] = jnp.maximum(x, 0.0)
        return 0
    lax.fori_loop(0, nrows, row, 0)


def _zero_buf(z_ref, nrows):
    def row(i, _):
        for j in range(PW // L):
            z_ref[i, pl.ds(j * L, L)] = jnp.zeros((L,), _f32)
        return 0
    lax.fori_loop(0, nrows, row, 0)


def _zero_stripe(zb_ref, tbl_ref, t, rows):
    nch = A // rows
    for jj in range(-(-nch // NT)):
        cid = t + jj * NT

        @pl.when(cid < nch)
        def _():
            pltpu.sync_copy(zb_ref,
                            tbl_ref.at[pl.ds(pl.multiple_of(cid * rows, 8), rows)])


def _dump_stripe(tbl_ref, buf_ref, out_at, t, rows):
    nch = A // rows
    for jj in range(-(-nch // NT)):
        cid = t + jj * NT

        @pl.when(cid < nch)
        def _():
            off = pl.multiple_of(cid * rows, 8)
            pltpu.sync_copy(tbl_ref.at[pl.ds(off, rows)], buf_ref)
            pltpu.sync_copy(buf_ref, out_at.at[pl.ds(off, rows)])


BSS = 160
_NCH_ALL = E // BSS
_NCH_S = -(-_NCH_ALL // NT)


@functools.partial(
    pl.kernel,
    out_type=jax.ShapeDtypeStruct((NP, A, PW), _f32),
    mesh=_mesh,
    scratch_types=[
        pltpu.VMEM((BSS,), jnp.int32),
        pltpu.VMEM((BSS,), jnp.int32),
        pltpu.VMEM((BSS, PW), _f32),
        pltpu.VMEM((BSS, PW), _f32),
        pltpu.VMEM((BC, PW), _f32),
        pltpu.VMEM_SHARED((A, PW), _f32),
        pltpu.SemaphoreType.DMA,
        pltpu.SemaphoreType.DMA,
        pltpu.SemaphoreType.DMA,
        pltpu.SemaphoreType.DMA,
        pltpu.SemaphoreType.DMA,
        pltpu.SemaphoreType.DMA,
    ],
)
def _sc_scatter(hw3, src, agg3,
                si0, si1, rb0, rb1, zb, tbl,
                smi0, smi1, smr0, smr1, smw0, smw1):
    c = lax.axis_index("c")
    t = lax.axis_index("s")
    _zero_buf(zb, BC)
    sib = (si0, si1)
    rbb = (rb0, rb1)
    smi = (smi0, smi1)
    smr = (smr0, smr1)
    smw = (smw0, smw1)

    for rnd in range(2):
        piece = 2 * c + rnd
        active = piece < NP

        @pl.when(active)
        def _():
            _zero_stripe(zb, tbl, t, BC)
        plsc.subcore_barrier()

        @pl.when(active)
        def _():
            def cid(k):
                return t + k * NT

            def base(k):
                return pl.multiple_of(cid(k) * BSS, 8)

            def load(k, bank):
                pltpu.async_copy(hw3.at[piece, pl.ds(base(k), BSS)],
                                 rbb[bank], smr[bank])
                pltpu.async_copy(src.at[pl.ds(base(k), BSS)], sib[bank],
                                 smi[bank])

            def wait_load(k, bank):
                pltpu.make_async_copy(hw3.at[piece, pl.ds(base(k), BSS)],
                                      rbb[bank], smr[bank]).wait()
                pltpu.make_async_copy(src.at[pl.ds(base(k), BSS)], sib[bank],
                                      smi[bank]).wait()

            load(0, 0)
            load(1, 1)

            def step(k, bank):
                @pl.when(cid(k) < _NCH_ALL)
                def _():
                    wait_load(k, bank)
                    pltpu.async_copy(rbb[bank], tbl.at[sib[bank]], smw[bank],
                                     add=True)

            def drain_issue(k, bank):
                @pl.when(cid(k) < _NCH_ALL)
                def _():
                    pltpu.make_async_copy(rbb[bank], tbl.at[sib[bank]],
                                          smw[bank]).wait()

                    @pl.when(cid(k + 2) < _NCH_ALL)
                    def _():
                        load(k + 2, bank)

            def body(j, _):
                for bank in range(2):
                    step(2 * j + bank, bank)
                for bank in range(2):
                    drain_issue(2 * j + bank, bank)
                return 0

            lax.fori_loop(0, _NCH_S // 2, body, 0)
            if _NCH_S % 2 == 1:
                k = _NCH_S - 1

                @pl.when(cid(k) < _NCH_ALL)
                def _():
                    wait_load(k, 0)
                    pltpu.sync_copy(rbb[0], tbl.at[sib[0]], add=True)
        plsc.subcore_barrier()

        @pl.when(active)
        def _():
            _dump_stripe(tbl, zb, agg3.at[piece], t, BC)
            _zero_buf(zb, BC)
        plsc.subcore_barrier()


def _combine_scratch(n_idx, bc):
    return ([pltpu.VMEM((bc,), jnp.int32) for _ in range(2 * n_idx)] +
            [pltpu.VMEM((bc, PW), _f32) for _ in range(8)] +
            [pltpu.SemaphoreType.DMA for _ in range(6)])


def _combine_pipeline(nch, bc, bofs, piece, hw3, src, rev, hinit3, agg3,
                      sb, rb, hib, agb, rvb, ob, smi, smg, sms,
                      store_fn=None):
    agg_t = agg3.at[piece]
    hw_t = hw3.at[piece]

    def prefetch(k, bank):
        b = bofs(k)
        pltpu.async_copy(src.at[pl.ds(b, bc)], sb[bank], smi[bank])
        pltpu.async_copy(rev.at[pl.ds(b, bc)], rb[bank], smi[bank])

    def wait_prefetch(k, bank):
        b = bofs(k)
        pltpu.make_async_copy(src.at[pl.ds(b, bc)], sb[bank], smi[bank]).wait()
        pltpu.make_async_copy(rev.at[pl.ds(b, bc)], rb[bank], smi[bank]).wait()

    def issue(k, bank):
        wait_prefetch(k, bank)
        b = bofs(k)
        pltpu.async_copy(hinit3.at[piece, pl.ds(b, bc)], hib[bank], smg[bank])
        pltpu.async_copy(agg_t.at[sb[bank]], agb[bank], smg[bank])
        pltpu.async_copy(hw_t.at[rb[bank]], rvb[bank], smg[bank])

    def wait_gathers(k, bank):
        b = bofs(k)
        pltpu.make_async_copy(hinit3.at[piece, pl.ds(b, bc)], hib[bank],
                              smg[bank]).wait()
        pltpu.make_async_copy(agg_t.at[sb[bank]], agb[bank], smg[bank]).wait()
        pltpu.make_async_copy(hw_t.at[rb[bank]], rvb[bank], smg[bank]).wait()

    def finish(k, bank):
        wait_gathers(k, bank)

        @pl.when(k + 2 < nch)
        def _():
            prefetch(k + 2, bank)

        store_fn(k, bank)

        @pl.when(k + 2 < nch)
        def _():
            issue(k + 2, bank)

    prefetch(0, 0)
    prefetch(1, 1)
    issue(0, 0)
    issue(1, 1)

    def body(j, _):
        finish(2 * j, 0)
        finish(2 * j + 1, 1)
        return 0

    lax.fori_loop(0, nch // 2, body, 0)
    if nch % 2 == 1:
        finish(nch - 1, 0)


@functools.partial(
    pl.kernel,
    out_type=jax.ShapeDtypeStruct((NP, E, PW), _f32),
    mesh=_mesh,
    scratch_types=_combine_scratch(2, BCC),
)
def _sc_combine(hw3, src, rev, hinit3, agg3, h3,
                sb0, sb1, rb0, rb1,
                hib0, hib1, agb0, agb1, rvb0, rvb1, ob0, ob1,
                smi0, smi1, smg0, smg1, sms0, sms1):
    c = lax.axis_index("c")
    t = lax.axis_index("s")
    w = c * NT + t
    ebase = w * EW
    sb, rb = (sb0, sb1), (rb0, rb1)
    hib, agb, rvb, ob = (hib0, hib1), (agb0, agb1), (rvb0, rvb1), (ob0, ob1)
    smi, smg, sms = (smi0, smi1), (smg0, smg1), (sms0, sms1)

    def bofs(k):
        return ebase + pl.multiple_of(jnp.minimum(k * BCC, EW - BCC), 8)

    for piece in range(NP):
        out_t = h3.at[piece]

        def store(k, bank, out_t=out_t):
            b = bofs(k)

            @pl.when(k >= 2)
            def _():
                pltpu.make_async_copy(ob[bank], out_t.at[pl.ds(b, BCC)],
                                      sms[bank]).wait()

            _relu_ab_minus_c(hib[bank], agb[bank], rvb[bank], ob[bank], BCC)
            pltpu.async_copy(ob[bank], out_t.at[pl.ds(b, BCC)], sms[bank])

        _combine_pipeline(NCC, BCC, bofs, piece, hw3, src, rev, hinit3, agg3,
                          sb, rb, hib, agb, rvb, ob, smi, smg, sms,
                          store_fn=store)
        for bank in range(2):
            pltpu.make_async_copy(ob[bank], out_t.at[pl.ds(ebase, BCC)],
                                  sms[bank]).wait()


@functools.partial(
    pl.kernel,
    out_type=jax.ShapeDtypeStruct((NP, A, PW), _f32),
    mesh=_mesh,
    scratch_types=(
        [pltpu.VMEM((BC,), jnp.int32) for _ in range(6)] +
        [pltpu.VMEM((BC, PW), _f32) for _ in range(8)] +
        [pltpu.VMEM((BC, PW), _f32),
         pltpu.VMEM_SHARED((A, PW), _f32)] +
        [pltpu.SemaphoreType.DMA for _ in range(10)]
    ),
)
def _sc_combine_final(hw3, src, rev, dst, hinit3, agg3, m3,
                      sb0, sb1, rb0, rb1, db0, db1,
                      hib0, hib1, agb0, agb1, rvb0, rvb1, ob0, ob1,
                      zb, tbl,
                      smi0, smi1, smg0, smg1, sms0, sms1, smd0, smd1,
                      smm0, smm1):
    c = lax.axis_index("c")
    t = lax.axis_index("s")
    _zero_buf(zb, BC)
    sb, rb, db = (sb0, sb1), (rb0, rb1), (db0, db1)
    hib, agb, rvb, ob = (hib0, hib1), (agb0, agb1), (rvb0, rvb1), (ob0, ob1)
    smi, smg, sms = (smi0, smi1), (smg0, smg1), (sms0, sms1)
    smd, smm = (smd0, smd1), (smm0, smm1)
    nch = EPT // BC

    for rnd in range(2):
        piece = 2 * c + rnd
        active = piece < NP

        @pl.when(active)
        def _():
            _zero_stripe(zb, tbl, t, BC)
        plsc.subcore_barrier()

        @pl.when(active)
        def _():
            ebase = t * EPT

            def bofs(k):
                return ebase + k * BC

            def store(k, bank):
                b = bofs(k)

                @pl.when(k >= 2)
                def _():
                    pltpu.make_async_copy(ob[bank], tbl.at[db[bank]],
                                          smm[bank]).wait()

                pltpu.async_copy(dst.at[pl.ds(b, BC)], db[bank], smd[bank])
                _relu_ab_minus_c(hib[bank], agb[bank], rvb[bank], ob[bank], BC)
                pltpu.make_async_copy(dst.at[pl.ds(b, BC)], db[bank],
                                      smd[bank]).wait()
                pltpu.async_copy(ob[bank], tbl.at[db[bank]], smm[bank],
                                 add=True)

            _combine_pipeline(nch, BC, bofs, piece, hw3, src, rev, hinit3,
                              agg3, sb, rb, hib, agb, rvb, ob, smi, smg, sms,
                              store_fn=store)
            for bank in range(2):
                pltpu.make_async_copy(ob[bank], tbl.at[db[bank]],
                                      smm[bank]).wait()
        plsc.subcore_barrier()

        @pl.when(active)
        def _():
            _dump_stripe(tbl, zb, m3.at[piece], t, BC)
            _zero_buf(zb, BC)
        plsc.subcore_barrier()


@functools.partial(
    pl.kernel,
    out_type=jax.ShapeDtypeStruct((NP, E, PW), _f32),
    mesh=_mesh,
    scratch_types=(
        [pltpu.VMEM((BCC,), jnp.int32) for _ in range(2)] +
        [pltpu.VMEM((BCC, PW), _f32) for _ in range(6)] +
        [pltpu.SemaphoreType.DMA for _ in range(6)]
    ),
)
def _sc_hinit(p3, q3, src, out,
              sb0, sb1, pb0, pb1, qb0, qb1, ob0, ob1,
              smi0, smi1, smg0, smg1, sms0, sms1):
    c = lax.axis_index("c")
    t = lax.axis_index("s")
    w = c * NT + t
    ebase = w * EW
    nch = NCC
    sb = (sb0, sb1)
    pb, qb, ob = (pb0, pb1), (qb0, qb1), (ob0, ob1)
    smi, smg, sms = (smi0, smi1), (smg0, smg1), (sms0, sms1)

    def bofs(k):
        return ebase + pl.multiple_of(jnp.minimum(k * BCC, EW - BCC), 8)

    for piece in range(NP):
        p_t = p3.at[piece]
        out_t = out.at[piece]

        def prefetch(k, bank):
            pltpu.async_copy(src.at[pl.ds(bofs(k), BCC)], sb[bank],
                             smi[bank])

        def issue(k, bank):
            b = bofs(k)
            pltpu.make_async_copy(src.at[pl.ds(b, BCC)], sb[bank],
                                  smi[bank]).wait()
            pltpu.async_copy(p_t.at[sb[bank]], pb[bank], smg[bank])
            pltpu.async_copy(q3.at[piece, pl.ds(b, BCC)], qb[bank], smg[bank])

        def finish(k, bank):
            b = bofs(k)
            pltpu.make_async_copy(p_t.at[sb[bank]], pb[bank], smg[bank]).wait()
            pltpu.make_async_copy(q3.at[piece, pl.ds(b, BCC)], qb[bank],
                                  smg[bank]).wait()

            @pl.when(k + 2 < nch)
            def _():
                prefetch(k + 2, bank)

            @pl.when(k >= 2)
            def _():
                pltpu.make_async_copy(ob[bank], out_t.at[pl.ds(b, BCC)],
                                      sms[bank]).wait()

            def row(i, _):
                for j in range(PW // L):
                    s = pl.ds(j * L, L)
                    ob[bank][i, s] = jnp.maximum(pb[bank][i, s]
                                                 + qb[bank][i, s], 0.0)
                return 0
            lax.fori_loop(0, BCC, row, 0)
            pltpu.async_copy(ob[bank], out_t.at[pl.ds(b, BCC)], sms[bank])

            @pl.when(k + 2 < nch)
            def _():
                issue(k + 2, bank)

        prefetch(0, 0)
        prefetch(1, 1)
        issue(0, 0)
        issue(1, 1)

        def body(j, _):
            finish(2 * j, 0)
            finish(2 * j + 1, 1)
            return 0

        lax.fori_loop(0, nch // 2, body, 0)
        if nch % 2 == 1:
            finish(nch - 1, 0)
        for bank in range(2):
            pltpu.make_async_copy(ob[bank], out_t.at[pl.ds(ebase, BCC)],
                                  sms[bank]).wait()


def _pad2(w, rows, cols):
    return jnp.pad(w, ((0, rows - w.shape[0]), (0, cols - w.shape[1])))


def kernel(atom_feats, bond_feats, src, dst, rev_idx, n_atoms,
           W_i, W_m0, W_m1, W_m2, W_a, b_a, W_f1, b_f1, W_f2, b_f2):
    del n_atoms
    src = src.astype(jnp.int32)
    dst = dst.astype(jnp.int32)
    rev_idx = rev_idx.astype(jnp.int32)

    af_pad = jnp.pad(atom_feats, ((0, 0), (0, 80 - 73)))
    bf_pad = jnp.pad(bond_feats, ((0, 0), (0, 16 - 13)))
    wi_top = _pad2(W_i[:73], 80, HP)
    wi_bot = _pad2(W_i[73:], 16, HP)
    wm = [_pad2(w, HP, HP) for w in (W_m0, W_m1, W_m2)]
    wa = jnp.concatenate([_pad2(W_a[:73], 80, HP), _pad2(W_a[73:], HP, HP)],
                         axis=0)
    ba = _pad2(b_a[None, :], 1, HP)
    wf1 = _pad2(W_f1, HP, HP)
    bf1 = _pad2(b_f1[None, :], 1, HP)
    wf2 = _pad2(W_f2, HP, 128)
    bf2 = _pad2(b_f2[None, :], 1, 128)

    p3 = _mm_pieces_out(af_pad, wi_top, 5000)
    q3 = _mm_pieces_out(bf_pad, wi_bot, 4000)
    hinit3 = _sc_hinit(p3, q3, src)

    h3 = hinit3
    for li in range(3):
        hw3 = _mm_pieces_both(h3, wm[li], 8000)
        agg3 = _sc_scatter(hw3, src)
        if li < 2:
            h3 = _sc_combine(hw3, src, rev_idx, hinit3, agg3)
        else:
            m3 = _sc_combine_final(hw3, src, rev_idx, dst, hinit3, agg3)

    out_pad = _readout(af_pad, m3, wa, ba, wf1, bf1, wf2, bf2, 5000)
    return out_pad[:, :3]
